# baseline (speedup 1.0000x reference)
import functools
import math

import jax
import jax.numpy as jnp
from jax import lax
from jax.experimental import pallas as pl
from jax.experimental.pallas import tpu as pltpu
from jax.experimental.pallas import tpu_sc as plsc

DIM = 64
SCALE = math.sqrt(DIM)
LANES = 16
NC, NS = 2, 16
NW = NC * NS

XR = 2
SEQ = 200
NBUF = 2
G_SPLITS = ((0, 128), (128, 72))


def _make_kernel(NROWS):
    assert NROWS % (NW * XR * NBUF) == 0
    n_chunks = NROWS // (NW * XR)
    mesh = plsc.VectorSubcoreMesh(core_axis_name="c", subcore_axis_name="s")

    @functools.partial(
        pl.kernel,
        out_type=jax.ShapeDtypeStruct((NROWS, SEQ, DIM), jnp.float32),
        mesh=mesh,
        scratch_types=[
            pltpu.VMEM((NBUF, XR, SEQ), jnp.int32),
            pltpu.VMEM((NBUF, XR, SEQ, DIM), jnp.float32),
            [pltpu.SemaphoreType.DMA] * NBUF,
            [pltpu.SemaphoreType.DMA] * NBUF,
        ],
        compiler_params=pltpu.CompilerParams(use_tc_tiling_on_sc=False),
    )
    def emb_kernel(x_hbm, w_hbm, out_hbm, idx_v, rows_v, gsem, ssem):
        wid = lax.axis_index("s") * NC + lax.axis_index("c")
        chunk0 = wid * n_chunks

        def fire_gathers(ci, b):
            row = (chunk0 + ci) * XR
            pltpu.sync_copy(x_hbm.at[pl.ds(row, XR)], idx_v.at[b])
            for i in range(XR):
                for off, n in G_SPLITS:
                    pltpu.async_copy(
                        w_hbm.at[idx_v.at[b].at[i, pl.ds(off, n)]],
                        rows_v.at[b].at[i].at[pl.ds(off, n)],
                        gsem[b],
                    )

        def wait_gathers(b):
            for i in range(XR):
                for off, n in G_SPLITS:
                    pltpu.make_async_copy(
                        w_hbm.at[idx_v.at[b].at[i, pl.ds(off, n)]],
                        rows_v.at[b].at[i].at[pl.ds(off, n)],
                        gsem[b],
                    ).wait()

        def fire_store(ci, b):
            row = (chunk0 + ci) * XR
            pltpu.async_copy(rows_v.at[b], out_hbm.at[pl.ds(row, XR)], ssem[b])

        def wait_store(ci, b):
            row = (chunk0 + ci) * XR
            pltpu.make_async_copy(
                rows_v.at[b], out_hbm.at[pl.ds(row, XR)], ssem[b]
            ).wait()

        def scale(b):
            def scale_body(r):
                for i in range(XR):
                    for j in range(DIM // LANES):
                        sl = pl.ds(j * LANES, LANES)
                        rows_v[b, i, r, sl] = rows_v[b, i, r, sl] * SCALE

            plsc.parallel_loop(0, SEQ, 1, unroll=8)(scale_body)

        fire_gathers(0, 0)

        def super_body(s, _):
            for b in range(NBUF):
                ci = s * NBUF + b
                nci = ci + 1
                nb = (b + 1) % NBUF

                @pl.when(jnp.logical_and(nci >= NBUF, nci < n_chunks))
                def _():
                    wait_store(nci - NBUF, nb)

                @pl.when(nci < n_chunks)
                def _():
                    fire_gathers(nci, nb)

                wait_gathers(b)
                scale(b)
                fire_store(ci, b)
            return ()

        lax.fori_loop(0, n_chunks // NBUF, super_body, ())

        for b in range(NBUF):
            wait_store(n_chunks - NBUF + b, b)

    return emb_kernel


@jax.jit
def kernel(x, W):
    return _make_kernel(x.shape[0])(x, W)
